# baseline (speedup 1.0000x reference)
import jax
import jax.numpy as jnp
from jax.experimental import pallas as pl
from jax.experimental.pallas import tpu as pltpu

E = 1_600_000
N = 50_000
D_EDGE = 16
D_NODE = 128
D_IN = 272
D_OUT = 128

G = 125
BE = E // G
BN = N // G


def _body(edge_ref, node_ref, glob_ref, w_ref, b_ref, out_ref, acc_e, acc_n):
    i = pl.program_id(0)

    @pl.when(i == 0)
    def _init():
        acc_e[...] = jnp.zeros_like(acc_e)
        acc_n[...] = jnp.zeros_like(acc_n)

    acc_e[...] += jnp.sum(edge_ref[...], axis=0, keepdims=True)
    acc_n[...] += jnp.sum(node_ref[...], axis=0, keepdims=True)

    @pl.when(i == G - 1)
    def _final():
        x = jnp.concatenate(
            [acc_e[...] * (1.0 / E), acc_n[...] * (1.0 / N), glob_ref[...]],
            axis=1,
        )
        out_ref[...] = jax.lax.dot_general(
            x, w_ref[...], (((1,), (0,)), ((), ())),
            preferred_element_type=jnp.float32,
        ) + b_ref[...]


def kernel(edge_attrs, node_attrs, global_attr, W, b):
    glob2 = global_attr.reshape(1, D_NODE)
    b2 = b.reshape(1, D_OUT)
    out = pl.pallas_call(
        _body,
        grid=(G,),
        in_specs=[
            pl.BlockSpec((BE, D_EDGE), lambda i: (i, 0)),
            pl.BlockSpec((BN, D_NODE), lambda i: (i, 0)),
            pl.BlockSpec((1, D_NODE), lambda i: (0, 0)),
            pl.BlockSpec((D_IN, D_OUT), lambda i: (0, 0)),
            pl.BlockSpec((1, D_OUT), lambda i: (0, 0)),
        ],
        out_specs=pl.BlockSpec((1, D_OUT), lambda i: (0, 0)),
        out_shape=jax.ShapeDtypeStruct((1, D_OUT), jnp.float32),
        scratch_shapes=[
            pltpu.VMEM((1, D_EDGE), jnp.float32),
            pltpu.VMEM((1, D_NODE), jnp.float32),
        ],
    )(edge_attrs, node_attrs, glob2, W, b2)
    return out.reshape(D_OUT)


# speedup vs baseline: 1.0014x; 1.0014x over previous
import jax
import jax.numpy as jnp
from jax.experimental import pallas as pl
from jax.experimental.pallas import tpu as pltpu

E = 1_600_000
N = 50_000
D_EDGE = 16
D_NODE = 128
D_OUT = 128

EV = E // 8
G = 125
BEV = EV // G
BN = N // G


def _body(edge_ref, node_ref, glob_ref, w_ref, b_ref, out_ref, acc):
    i = pl.program_id(0)

    @pl.when(i == 0)
    def _init():
        acc[...] = jnp.zeros_like(acc)

    acc[0:1, :] += jnp.sum(edge_ref[...], axis=0, keepdims=True)
    acc[1:2, :] += jnp.sum(node_ref[...], axis=0, keepdims=True)

    @pl.when(i == G - 1)
    def _final():
        x = jnp.concatenate(
            [acc[0:1, :] * (1.0 / E), acc[1:2, :] * (1.0 / N), glob_ref[...]],
            axis=1,
        )
        out_ref[...] = jax.lax.dot_general(
            x, w_ref[...], (((1,), (0,)), ((), ())),
            preferred_element_type=jnp.float32,
        ) + b_ref[...]


def kernel(edge_attrs, node_attrs, global_attr, W, b):
    edge_view = edge_attrs.reshape(EV, 128)
    w_big = jnp.concatenate(
        [jnp.tile(W[:D_EDGE, :], (8, 1)), W[D_EDGE:, :]], axis=0
    )
    glob2 = global_attr.reshape(1, D_NODE)
    b2 = b.reshape(1, D_OUT)
    out = pl.pallas_call(
        _body,
        grid=(G,),
        in_specs=[
            pl.BlockSpec((BEV, 128), lambda i: (i, 0)),
            pl.BlockSpec((BN, D_NODE), lambda i: (i, 0)),
            pl.BlockSpec((1, D_NODE), lambda i: (0, 0)),
            pl.BlockSpec((384, D_OUT), lambda i: (0, 0)),
            pl.BlockSpec((1, D_OUT), lambda i: (0, 0)),
        ],
        out_specs=pl.BlockSpec((1, D_OUT), lambda i: (0, 0)),
        out_shape=jax.ShapeDtypeStruct((1, D_OUT), jnp.float32),
        scratch_shapes=[
            pltpu.VMEM((2, D_NODE), jnp.float32),
        ],
    )(edge_view, node_attrs, glob2, w_big, b2)
    return out.reshape(D_OUT)


# speedup vs baseline: 7.1552x; 7.1451x over previous
import jax
import jax.numpy as jnp
from jax.experimental import pallas as pl
from jax.experimental.pallas import tpu as pltpu

E = 1_600_000
N = 50_000
D_EDGE = 16
D_NODE = 128
D_IN = 272
D_OUT = 128

G = 125
BL = E // G
BN = N // G


def _body(edge_ref, node_ref, glob_ref, w_ref, b_ref, out_ref, acc_e, acc_n):
    i = pl.program_id(0)

    @pl.when(i == 0)
    def _init():
        acc_e[...] = jnp.zeros_like(acc_e)
        acc_n[...] = jnp.zeros_like(acc_n)

    e = edge_ref[...].reshape(D_EDGE, BL // 128, 128)
    acc_e[...] += jnp.sum(e, axis=1)
    acc_n[...] += jnp.sum(node_ref[...], axis=0, keepdims=True)

    @pl.when(i == G - 1)
    def _final():
        e16 = jnp.sum(acc_e[...], axis=1, keepdims=True)
        x = jnp.concatenate(
            [e16.reshape(1, D_EDGE) * (1.0 / E),
             acc_n[...] * (1.0 / N),
             glob_ref[...]],
            axis=1,
        )
        out_ref[...] = jax.lax.dot_general(
            x, w_ref[...], (((1,), (0,)), ((), ())),
            preferred_element_type=jnp.float32,
        ) + b_ref[...]


def kernel(edge_attrs, node_attrs, global_attr, W, b):
    edge_view = edge_attrs.T
    glob2 = global_attr.reshape(1, D_NODE)
    b2 = b.reshape(1, D_OUT)
    out = pl.pallas_call(
        _body,
        grid=(G,),
        in_specs=[
            pl.BlockSpec((D_EDGE, BL), lambda i: (0, i)),
            pl.BlockSpec((BN, D_NODE), lambda i: (i, 0)),
            pl.BlockSpec((1, D_NODE), lambda i: (0, 0)),
            pl.BlockSpec((D_IN, D_OUT), lambda i: (0, 0)),
            pl.BlockSpec((1, D_OUT), lambda i: (0, 0)),
        ],
        out_specs=pl.BlockSpec((1, D_OUT), lambda i: (0, 0)),
        out_shape=jax.ShapeDtypeStruct((1, D_OUT), jnp.float32),
        scratch_shapes=[
            pltpu.VMEM((D_EDGE, D_NODE), jnp.float32),
            pltpu.VMEM((1, D_NODE), jnp.float32),
        ],
    )(edge_view, node_attrs, glob2, W, b2)
    return out.reshape(D_OUT)


# speedup vs baseline: 12.8523x; 1.7962x over previous
import jax
import jax.numpy as jnp
from jax.experimental import pallas as pl
from jax.experimental.pallas import tpu as pltpu

E = 1_600_000
N = 50_000
D_EDGE = 16
D_NODE = 128
D_IN = 272
D_OUT = 128

G = 50
BL = E // G
BN = N // G


def _body(edge_ref, node_ref, glob_ref, w_ref, b_ref, out_ref, acc_e, acc_n):
    i = pl.program_id(0)

    @pl.when(i == 0)
    def _init():
        acc_e[...] = jnp.zeros_like(acc_e)
        acc_n[...] = jnp.zeros_like(acc_n)

    acc_e[...] += edge_ref[...]
    acc_n[...] += node_ref[...]

    @pl.when(i == G - 1)
    def _final():
        e16 = jnp.sum(acc_e[...].reshape(D_EDGE, BL // 128, 128), axis=(1, 2))
        n128 = jnp.sum(acc_n[...], axis=0, keepdims=True)
        x = jnp.concatenate(
            [e16.reshape(1, D_EDGE) * (1.0 / E),
             n128 * (1.0 / N),
             glob_ref[...]],
            axis=1,
        )
        out_ref[...] = jax.lax.dot_general(
            x, w_ref[...], (((1,), (0,)), ((), ())),
            preferred_element_type=jnp.float32,
        ) + b_ref[...]


def kernel(edge_attrs, node_attrs, global_attr, W, b):
    edge_view = edge_attrs.T
    glob2 = global_attr.reshape(1, D_NODE)
    b2 = b.reshape(1, D_OUT)
    out = pl.pallas_call(
        _body,
        grid=(G,),
        in_specs=[
            pl.BlockSpec((D_EDGE, BL), lambda i: (0, i)),
            pl.BlockSpec((BN, D_NODE), lambda i: (i, 0)),
            pl.BlockSpec((1, D_NODE), lambda i: (0, 0)),
            pl.BlockSpec((D_IN, D_OUT), lambda i: (0, 0)),
            pl.BlockSpec((1, D_OUT), lambda i: (0, 0)),
        ],
        out_specs=pl.BlockSpec((1, D_OUT), lambda i: (0, 0)),
        out_shape=jax.ShapeDtypeStruct((1, D_OUT), jnp.float32),
        scratch_shapes=[
            pltpu.VMEM((D_EDGE, BL), jnp.float32),
            pltpu.VMEM((BN, D_NODE), jnp.float32),
        ],
    )(edge_view, node_attrs, glob2, W, b2)
    return out.reshape(D_OUT)


# speedup vs baseline: 13.8290x; 1.0760x over previous
import jax
import jax.numpy as jnp
from jax.experimental import pallas as pl
from jax.experimental.pallas import tpu as pltpu

E = 1_600_000
N = 50_000
D_EDGE = 16
D_NODE = 128
D_IN = 272
D_OUT = 128

G = 50
BL = E // G
BN = N // G
AE = 1280
AN = 40


def _body(edge_ref, node_ref, glob_ref, w_ref, b_ref, out_ref, acc_e, acc_n):
    i = pl.program_id(0)

    @pl.when(i == 0)
    def _init():
        acc_e[...] = jnp.zeros_like(acc_e)
        acc_n[...] = jnp.zeros_like(acc_n)

    e = edge_ref[...]
    ae = acc_e[...]
    for k in range(BL // AE):
        ae = ae + e[:, k * AE:(k + 1) * AE]
    acc_e[...] = ae
    nb = node_ref[...]
    an = acc_n[...]
    for k in range(BN // AN):
        an = an + nb[k * AN:(k + 1) * AN, :]
    acc_n[...] = an

    @pl.when(i == G - 1)
    def _final():
        e16 = jnp.sum(acc_e[...].reshape(D_EDGE, AE // 128, 128), axis=(1, 2))
        n128 = jnp.sum(acc_n[...], axis=0, keepdims=True)
        x = jnp.concatenate(
            [e16.reshape(1, D_EDGE) * (1.0 / E),
             n128 * (1.0 / N),
             glob_ref[...]],
            axis=1,
        )
        out_ref[...] = jax.lax.dot_general(
            x, w_ref[...], (((1,), (0,)), ((), ())),
            preferred_element_type=jnp.float32,
        ) + b_ref[...]


def kernel(edge_attrs, node_attrs, global_attr, W, b):
    edge_view = edge_attrs.T
    glob2 = global_attr.reshape(1, D_NODE)
    b2 = b.reshape(1, D_OUT)
    out = pl.pallas_call(
        _body,
        grid=(G,),
        in_specs=[
            pl.BlockSpec((D_EDGE, BL), lambda i: (0, i)),
            pl.BlockSpec((BN, D_NODE), lambda i: (i, 0)),
            pl.BlockSpec((1, D_NODE), lambda i: (0, 0)),
            pl.BlockSpec((D_IN, D_OUT), lambda i: (0, 0)),
            pl.BlockSpec((1, D_OUT), lambda i: (0, 0)),
        ],
        out_specs=pl.BlockSpec((1, D_OUT), lambda i: (0, 0)),
        out_shape=jax.ShapeDtypeStruct((1, D_OUT), jnp.float32),
        scratch_shapes=[
            pltpu.VMEM((D_EDGE, AE), jnp.float32),
            pltpu.VMEM((AN, D_NODE), jnp.float32),
        ],
    )(edge_view, node_attrs, glob2, W, b2)
    return out.reshape(D_OUT)
